# Initial kernel scaffold; baseline (speedup 1.0000x reference)
import jax
import jax.numpy as jnp
from jax.experimental import pallas as pl


def kernel(y_pred_softmax, y_true):
    raise NotImplementedError("write your pallas kernel here")


# speedup vs baseline: 36.6139x; 36.6139x over previous
import jax
import jax.numpy as jnp
from jax import lax
from jax.experimental import pallas as pl
from jax.experimental.pallas import tpu as pltpu

_K = 2000
_TH = 0.001
_NIMG = 12
_H = 512
_W = 512
_NPIX = _H * _W


def _i32_bits(xf):
    v = jnp.full((8, 128), xf, jnp.float32)
    return jnp.max(lax.bitcast_convert_type(v, jnp.int32))


def _f32_val(xi):
    v = jnp.full((8, 128), xi, jnp.int32)
    return jnp.max(lax.bitcast_convert_type(v, jnp.float32))


def _nmin4(x):
    a = pltpu.roll(x, 1, 0)
    b = pltpu.roll(x, _H - 1, 0)
    c = pltpu.roll(x, 1, 1)
    d = pltpu.roll(x, _W - 1, 1)
    return jnp.minimum(jnp.minimum(a, b), jnp.minimum(c, d))


def _body(p_ref, yt_ref, out_ref, xs_ref, smin_ref, smax_ref, scnt_ref,
          acc_ref):
    ph = pl.program_id(0)
    i = pl.program_id(1)
    ch = i % 3 + 1
    p = p_ref[0, 0]

    @pl.when(ph == 0)
    def _phase0():
        @pl.when(i == 0)
        def _init():
            acc_ref[0] = 0.0
            acc_ref[1] = 0.0

        mn = jnp.min(p)
        mx = jnp.max(p)
        smin_ref[i] = mn
        smax_ref[0, i] = mx - mn
        x1 = jnp.maximum(p - _nmin4(p), 0.0)
        smax_ref[1, i] = jnp.max(x1)

        yt = yt_ref[0]
        e = (yt == ch).astype(jnp.int32)
        n1 = jnp.sum(e)
        c0 = jnp.where(n1 >= _NPIX, 0, jnp.minimum(n1, _K))
        nbmin = _nmin4(e)
        eb = e * (1 - nbmin)
        c1 = jnp.minimum(jnp.sum(eb), _K)
        scnt_ref[0, i] = c0
        scnt_ref[1, i] = c1

    @pl.when(ph == 1)
    def _phase1():
        for v in (0, 1):
            if v == 0:
                x = p - smin_ref[i]
            else:
                x = jnp.maximum(p - _nmin4(p), 0.0)
            xs_ref[...] = lax.bitcast_convert_type(x, jnp.int32)

            gmax = smax_ref[v, 0]
            for j in range(1, _NIMG):
                gmax = jnp.maximum(gmax, smax_ref[v, j])
            g = jnp.maximum(gmax, 1e-8)
            tau = _TH * g
            c = scnt_ref[v, i]
            rank_c = jnp.maximum(c, 1)
            hi0 = _i32_bits(smax_ref[v, i])

            def bis(_, carry):
                lo_k, hi_k, lo_c, hi_c = carry
                mid_k = lo_k + ((hi_k - lo_k + 1) >> 1)
                mid_c = lo_c + ((hi_c - lo_c + 1) >> 1)
                xsb = xs_ref[...]
                ck = jnp.sum((xsb >= mid_k).astype(jnp.int32))
                cc = jnp.sum((xsb >= mid_c).astype(jnp.int32))
                ok_k = ck >= _K
                ok_c = cc >= rank_c
                return (jnp.where(ok_k, mid_k, lo_k),
                        jnp.where(ok_k, hi_k, mid_k - 1),
                        jnp.where(ok_c, mid_c, lo_c),
                        jnp.where(ok_c, hi_c, mid_c - 1))

            lo_k, _, lo_c, _ = lax.fori_loop(
                0, 31, bis, (jnp.int32(0), hi0, jnp.int32(0), hi0))
            tk = _f32_val(lo_k)
            tc = _f32_val(lo_c)

            xv = lax.bitcast_convert_type(xs_ref[...], jnp.float32)
            m2 = jnp.maximum(tk, tau)
            m1 = jnp.maximum(tc, tau)
            mask2 = xv > m2
            mask1 = xv > m1
            a2 = jnp.sum(jnp.where(mask2, xv * xv, 0.0))
            c2 = jnp.sum(mask2.astype(jnp.int32)).astype(jnp.float32)
            a1 = jnp.sum(jnp.where(mask1, xv, 0.0))
            c1n = jnp.sum(mask1.astype(jnp.int32)).astype(jnp.float32)
            cf = c.astype(jnp.float32)
            s2 = (a2 + jnp.where(tk > tau, (_K - c2) * tk * tk, 0.0)) / (g * g)
            t_sum = jnp.where(
                c > 0,
                (a1 + jnp.where(tc > tau, (cf - c1n) * tc, 0.0)) / g,
                0.0)
            acc_ref[v] = acc_ref[v] + s2 - 2.0 * t_sum + cf

        @pl.when(i == _NIMG - 1)
        def _final():
            l0 = acc_ref[0] / (_NIMG * _K)
            l1 = acc_ref[1] / (_NIMG * _K)
            den = jnp.maximum(l0 + l1, 1e-8)
            out_ref[...] = jnp.full((1, 1), 2.0 * l0 * l1 / den, jnp.float32)


def _run(y_pred_softmax, y_true, interpret=False):
    out = pl.pallas_call(
        _body,
        grid=(2, _NIMG),
        in_specs=[
            pl.BlockSpec((1, 1, _H, _W), lambda ph, i: (i // 3, i % 3 + 1, 0, 0)),
            pl.BlockSpec((1, _H, _W), lambda ph, i: (i // 3, 0, 0)),
        ],
        out_specs=pl.BlockSpec((1, 1), lambda ph, i: (0, 0)),
        out_shape=jax.ShapeDtypeStruct((1, 1), jnp.float32),
        scratch_shapes=[
            pltpu.VMEM((_H, _W), jnp.int32),
            pltpu.SMEM((_NIMG,), jnp.float32),
            pltpu.SMEM((2, _NIMG), jnp.float32),
            pltpu.SMEM((2, _NIMG), jnp.int32),
            pltpu.SMEM((2,), jnp.float32),
        ],
        interpret=interpret,
    )(y_pred_softmax, y_true)
    return out[0, 0]


def kernel(y_pred_softmax, y_true):
    return _run(y_pred_softmax, y_true)
